# baseline (speedup 1.0000x reference)
import jax
import jax.numpy as jnp
from jax import lax
from jax.experimental import pallas as pl
from jax.experimental.pallas import tpu as pltpu
from jax.experimental.pallas import tpu_sc as plsc

_VOCAB = 100000
_EMB = 64
_CTX2 = 20
_HID = 512

_BLK = 2000
_NB = _VOCAB // _BLK


def _gather_body(idx_hbm, table_hbm, out_hbm, idx_v, rows_v, sem):
    wid = lax.axis_index("s") * 2 + lax.axis_index("c")

    @pl.when(wid == 0)
    def _():
        pltpu.sync_copy(idx_hbm, idx_v)
        v_lo = idx_v[pl.ds(0, 16)]
        v_hi = idx_v[pl.ds(4, 16)]
        copies = []
        for j in range(_CTX2):
            r = v_lo[j] if j < 16 else v_hi[j - 4]
            copies.append(pltpu.async_copy(
                table_hbm.at[pl.ds(r, 1)], rows_v.at[pl.ds(j, 1)], sem))
        for c in copies:
            c.wait()
        pltpu.sync_copy(rows_v, out_hbm)


def _sc_gather(idx, emb):
    mesh = plsc.VectorSubcoreMesh(core_axis_name="c", subcore_axis_name="s")
    return pl.kernel(
        _gather_body,
        out_type=jax.ShapeDtypeStruct((_CTX2, _EMB), jnp.float32),
        mesh=mesh,
        scratch_types=[
            pltpu.VMEM((_CTX2,), jnp.int32),
            pltpu.VMEM((_CTX2, _EMB), jnp.float32),
            pltpu.SemaphoreType.DMA,
        ],
    )(idx, emb)


def _mlp_body(x_ref, w1_ref, b1_ref, w2_ref, b2_ref, out_ref, h_ref):
    i = pl.program_id(0)

    @pl.when(i == 0)
    def _():
        h = lax.dot_general(
            x_ref[...], w1_ref[...], (((1,), (1,)), ((), ())),
            preferred_element_type=jnp.float32)
        h_ref[...] = jnp.maximum(h + b1_ref[...], 0.0)

    logits = lax.dot_general(
        h_ref[...], w2_ref[...], (((1,), (1,)), ((), ())),
        preferred_element_type=jnp.float32)
    out_ref[pl.ds(i, 1)] = (logits + b2_ref[0]).reshape(1, 1, _BLK)

    @pl.when(i == _NB - 1)
    def _():
        def _max_body(j, m):
            return jnp.maximum(m, jnp.max(out_ref[pl.ds(j, 1)]))

        m = lax.fori_loop(0, _NB, _max_body, jnp.float32(-jnp.inf))

        def _sum_body(j, s):
            return s + jnp.sum(jnp.exp(out_ref[pl.ds(j, 1)] - m))

        s = lax.fori_loop(0, _NB, _sum_body, jnp.float32(0.0))

        def _sub_body(j, carry):
            c = out_ref[pl.ds(j, 1)]
            out_ref[pl.ds(j, 1)] = c - m - jnp.log(jnp.broadcast_to(s, c.shape))
            return carry

        lax.fori_loop(0, _NB, _sub_body, 0)


def _mlp_logsoftmax(x, W1, b1, W2, b2):
    return pl.pallas_call(
        _mlp_body,
        grid=(_NB,),
        in_specs=[
            pl.BlockSpec((1, _CTX2 * _EMB), lambda i: (0, 0)),
            pl.BlockSpec((_HID, _CTX2 * _EMB), lambda i: (0, 0)),
            pl.BlockSpec((1, _HID), lambda i: (0, 0)),
            pl.BlockSpec((_BLK, _HID), lambda i: (i, 0)),
            pl.BlockSpec((1, 1, _BLK), lambda i: (i, 0, 0)),
        ],
        out_specs=pl.BlockSpec((_NB, 1, _BLK), lambda i: (0, 0, 0)),
        out_shape=jax.ShapeDtypeStruct((_NB, 1, _BLK), jnp.float32),
        scratch_shapes=[pltpu.VMEM((1, _HID), jnp.float32)],
    )(x, W1, b1, W2, b2)


def kernel(inp, emb, W1, b1, W2, b2):
    gathered = _sc_gather(inp.astype(jnp.int32), emb)
    x = gathered.reshape(1, _CTX2 * _EMB)
    out = _mlp_logsoftmax(x, W1, b1.reshape(1, _HID), W2,
                          b2.reshape(_NB, 1, _BLK))
    return out.reshape(1, _VOCAB)


# speedup vs baseline: 1.1600x; 1.1600x over previous
import jax
import jax.numpy as jnp
from jax import lax
from jax.experimental import pallas as pl
from jax.experimental.pallas import tpu as pltpu
from jax.experimental.pallas import tpu_sc as plsc

_VOCAB = 100000
_EMB = 64
_CTX2 = 20
_HID = 512

_BLK = 4000
_NB = _VOCAB // _BLK


def _gather_body(idx_hbm, table_hbm, out_hbm, idx_v, rows_v, sem):
    wid = lax.axis_index("s") * 2 + lax.axis_index("c")

    @pl.when(wid == 0)
    def _():
        pltpu.sync_copy(idx_hbm, idx_v)
        v_lo = idx_v[pl.ds(0, 16)]
        v_hi = idx_v[pl.ds(4, 16)]
        copies = []
        for j in range(_CTX2):
            r = v_lo[j] if j < 16 else v_hi[j - 4]
            copies.append(pltpu.async_copy(
                table_hbm.at[pl.ds(r, 1)], rows_v.at[pl.ds(j, 1)], sem))
        for c in copies:
            c.wait()
        pltpu.sync_copy(rows_v, out_hbm)


def _sc_gather(idx, emb):
    mesh = plsc.VectorSubcoreMesh(core_axis_name="c", subcore_axis_name="s")
    return pl.kernel(
        _gather_body,
        out_type=jax.ShapeDtypeStruct((_CTX2, _EMB), jnp.float32),
        mesh=mesh,
        scratch_types=[
            pltpu.VMEM((_CTX2,), jnp.int32),
            pltpu.VMEM((_CTX2, _EMB), jnp.float32),
            pltpu.SemaphoreType.DMA,
        ],
    )(idx, emb)


def _mlp_body(x_ref, w1_ref, b1_ref, w2_ref, b2_ref, out_ref, h_ref, m_ref):
    i = pl.program_id(0)

    @pl.when(i == 0)
    def _():
        h = lax.dot_general(
            x_ref[...], w1_ref[...], (((1,), (1,)), ((), ())),
            preferred_element_type=jnp.float32)
        h_ref[...] = jnp.maximum(h + b1_ref[...], 0.0).astype(jnp.bfloat16)
        m_ref[0] = jnp.float32(-jnp.inf)

    logits = lax.dot_general(
        h_ref[...], w2_ref[...].astype(jnp.bfloat16), (((1,), (1,)), ((), ())),
        preferred_element_type=jnp.float32)
    blk = logits + b2_ref[0]
    out_ref[pl.ds(i, 1)] = blk.reshape(1, 1, _BLK)
    m_ref[0] = jnp.maximum(m_ref[0], jnp.max(blk))

    @pl.when(i == _NB - 1)
    def _():
        m = m_ref[0]

        def _sum_body(j, s):
            return s + jnp.sum(jnp.exp(out_ref[pl.ds(j, 1)] - m))

        s = lax.fori_loop(0, _NB, _sum_body, jnp.float32(0.0))

        def _sub_body(j, carry):
            c = out_ref[pl.ds(j, 1)]
            out_ref[pl.ds(j, 1)] = c - m - jnp.log(jnp.broadcast_to(s, c.shape))
            return carry

        lax.fori_loop(0, _NB, _sub_body, 0)


def _mlp_logsoftmax(x, W1, b1, W2, b2):
    return pl.pallas_call(
        _mlp_body,
        grid=(_NB,),
        in_specs=[
            pl.BlockSpec((1, _CTX2 * _EMB), lambda i: (0, 0)),
            pl.BlockSpec((_HID, _CTX2 * _EMB), lambda i: (0, 0)),
            pl.BlockSpec((1, _HID), lambda i: (0, 0)),
            pl.BlockSpec((_BLK, _HID), lambda i: (i, 0)),
            pl.BlockSpec((1, 1, _BLK), lambda i: (i, 0, 0)),
        ],
        out_specs=pl.BlockSpec((_NB, 1, _BLK), lambda i: (0, 0, 0)),
        out_shape=jax.ShapeDtypeStruct((_NB, 1, _BLK), jnp.float32),
        scratch_shapes=[pltpu.VMEM((1, _HID), jnp.bfloat16),
                        pltpu.SMEM((1,), jnp.float32)],
    )(x, W1, b1, W2, b2)


def kernel(inp, emb, W1, b1, W2, b2):
    gathered = _sc_gather(inp.astype(jnp.int32), emb)
    x = gathered.reshape(1, _CTX2 * _EMB)
    out = _mlp_logsoftmax(x, W1, b1.reshape(1, _HID), W2,
                          b2.reshape(_NB, 1, _BLK))
    return out.reshape(1, _VOCAB)


# speedup vs baseline: 1.2007x; 1.0351x over previous
import jax
import jax.numpy as jnp
from jax import lax
from jax.experimental import pallas as pl
from jax.experimental.pallas import tpu as pltpu
from jax.experimental.pallas import tpu_sc as plsc

_VOCAB = 100000
_EMB = 64
_CTX2 = 20
_HID = 512

_BLK = 5000
_NB = _VOCAB // _BLK


def _gather_body(idx_hbm, table_hbm, out_hbm, idx_v, rows_v, sem):
    wid = lax.axis_index("s") * 2 + lax.axis_index("c")

    @pl.when(wid == 0)
    def _():
        pltpu.sync_copy(idx_hbm, idx_v)
        v_lo = idx_v[pl.ds(0, 16)]
        v_hi = idx_v[pl.ds(4, 16)]
        copies = []
        for j in range(_CTX2):
            r = v_lo[j] if j < 16 else v_hi[j - 4]
            copies.append(pltpu.async_copy(
                table_hbm.at[pl.ds(r, 1)], rows_v.at[pl.ds(j, 1)], sem))
        for c in copies:
            c.wait()
        pltpu.sync_copy(rows_v, out_hbm)


def _sc_gather(idx, emb):
    mesh = plsc.VectorSubcoreMesh(core_axis_name="c", subcore_axis_name="s")
    return pl.kernel(
        _gather_body,
        out_type=jax.ShapeDtypeStruct((_CTX2, _EMB), jnp.float32),
        mesh=mesh,
        scratch_types=[
            pltpu.VMEM((_CTX2,), jnp.int32),
            pltpu.VMEM((_CTX2, _EMB), jnp.float32),
            pltpu.SemaphoreType.DMA,
        ],
    )(idx, emb)


def _mlp_body(x_ref, w1_ref, b1_ref, w2_ref, b2_ref, out_ref, h_ref, ms_ref):
    i = pl.program_id(0)

    @pl.when(i == 0)
    def _():
        h = lax.dot_general(
            x_ref[...], w1_ref[...], (((1,), (1,)), ((), ())),
            preferred_element_type=jnp.float32)
        h_ref[...] = jnp.maximum(h + b1_ref[...], 0.0).astype(jnp.bfloat16)
        ms_ref[0] = jnp.float32(-jnp.inf)
        ms_ref[1] = jnp.float32(0.0)

    logits = lax.dot_general(
        h_ref[...], w2_ref[...].astype(jnp.bfloat16), (((1,), (1,)), ((), ())),
        preferred_element_type=jnp.float32)
    blk = logits + b2_ref[0]
    out_ref[pl.ds(i, 1)] = blk.reshape(1, 1, _BLK)
    m_old = ms_ref[0]
    m_new = jnp.maximum(m_old, jnp.max(blk))
    blk_sum = jnp.sum(jnp.exp(blk - m_new))
    scale = jnp.exp(jnp.broadcast_to(m_old - m_new, (1, 128)))[0, 0]
    ms_ref[0] = m_new
    ms_ref[1] = ms_ref[1] * scale + blk_sum

    @pl.when(i == _NB - 1)
    def _():
        m = ms_ref[0]
        s = ms_ref[1]

        def _sub_body(j, carry):
            c = out_ref[pl.ds(j, 1)]
            out_ref[pl.ds(j, 1)] = c - m - jnp.log(jnp.broadcast_to(s, c.shape))
            return carry

        lax.fori_loop(0, _NB, _sub_body, 0)


def _mlp_logsoftmax(x, W1, b1, W2, b2):
    return pl.pallas_call(
        _mlp_body,
        grid=(_NB,),
        in_specs=[
            pl.BlockSpec((1, _CTX2 * _EMB), lambda i: (0, 0)),
            pl.BlockSpec((_HID, _CTX2 * _EMB), lambda i: (0, 0)),
            pl.BlockSpec((1, _HID), lambda i: (0, 0)),
            pl.BlockSpec((_BLK, _HID), lambda i: (i, 0)),
            pl.BlockSpec((1, 1, _BLK), lambda i: (i, 0, 0)),
        ],
        out_specs=pl.BlockSpec((_NB, 1, _BLK), lambda i: (0, 0, 0)),
        out_shape=jax.ShapeDtypeStruct((_NB, 1, _BLK), jnp.float32),
        scratch_shapes=[pltpu.VMEM((1, _HID), jnp.bfloat16),
                        pltpu.SMEM((2,), jnp.float32)],
    )(x, W1, b1, W2, b2)


def kernel(inp, emb, W1, b1, W2, b2):
    gathered = _sc_gather(inp.astype(jnp.int32), emb)
    x = gathered.reshape(1, _CTX2 * _EMB)
    out = _mlp_logsoftmax(x, W1, b1.reshape(1, _HID), W2,
                          b2.reshape(_NB, 1, _BLK))
    return out.reshape(1, _VOCAB)
